# Initial kernel scaffold; baseline (speedup 1.0000x reference)
import functools
import math

import jax
import jax.numpy as jnp
from jax import lax
from jax.experimental import pallas as pl
from jax.experimental.pallas import tpu as pltpu
from jax.experimental.pallas import tpu_sc as plsc

B, T, D = 4, 4096, 1024
E, K = 64, 2
D_EXP, D_FF = 256, 256
N = B * T
NK = N * K
CAP = int(math.ceil(N * K / E * 1.25))
TB = 512
NT = N // TB
NWORK = 32
TPW = N // NWORK
DCH = 32
CCH = 16
TRASH = E * CAP
DISP_ROWS = E * CAP + 8


def _gelu(x):
    return 0.5 * x * (1.0 + lax.erf(x * 0.7071067811865476))


def _router_body(x_ref, wr_ref, wg_ref, ws1_ref, ws2_ref,
                 y_ref, sd0_ref, sd1_ref, sc0_ref, sc1_ref,
                 g0_ref, g1_ref, imp_ref, cnt_ref, carry_ref):
    t = pl.program_id(0)

    @pl.when(t == 0)
    def _():
        imp_ref[...] = jnp.zeros_like(imp_ref)
        cnt_ref[...] = jnp.zeros_like(cnt_ref)
        carry_ref[...] = jnp.zeros_like(carry_ref)

    x = x_ref[...]
    logits = jax.lax.dot_general(x, wr_ref[...],
                                 (((1,), (1,)), ((), ())),
                                 preferred_element_type=jnp.float32)
    m = jnp.max(logits, axis=1, keepdims=True)
    ex = jnp.exp(logits - m)
    s = ex / jnp.sum(ex, axis=1, keepdims=True)

    iota_e = lax.broadcasted_iota(jnp.int32, (TB, E), 1)
    v0 = jnp.max(s, axis=1)
    i0 = jnp.min(jnp.where(s == v0[:, None], iota_e, E), axis=1)
    s_m = jnp.where(iota_e == i0[:, None], -1.0, s)
    v1 = jnp.max(s_m, axis=1)
    i1 = jnp.min(jnp.where(s_m == v1[:, None], iota_e, E), axis=1)

    oh0 = (iota_e == i0[:, None]).astype(jnp.float32)
    oh1 = (iota_e == i1[:, None]).astype(jnp.float32)
    ohs = oh0 + oh1

    imp_ref[...] += jnp.sum(s, axis=0)[None, :]
    cnt_ref[...] += jnp.sum(ohs, axis=0)[None, :]

    ri = lax.broadcasted_iota(jnp.int32, (TB, TB), 0)
    ci = lax.broadcasted_iota(jnp.int32, (TB, TB), 1)
    tri = (ri > ci).astype(jnp.float32)
    cum = jax.lax.dot_general(tri, ohs, (((1,), (0,)), ((), ())),
                              preferred_element_type=jnp.float32)
    cum = cum + carry_ref[...]
    p0 = jnp.sum(cum * oh0, axis=1)
    p1 = jnp.sum(cum * oh1, axis=1)
    carry_ref[...] += jnp.sum(ohs, axis=0)[None, :]

    keep0 = p0 < float(CAP)
    keep1 = p1 < float(CAP)
    slot0 = i0 * CAP + p0.astype(jnp.int32)
    slot1 = i1 * CAP + p1.astype(jnp.int32)
    sd0_ref[0, 0, :] = jnp.where(keep0, slot0, TRASH)
    sd1_ref[0, 0, :] = jnp.where(keep1, slot1, TRASH)
    sc0_ref[0, 0, :] = jnp.where(keep0, slot0, i0 * CAP)
    sc1_ref[0, 0, :] = jnp.where(keep1, slot1, i1 * CAP)
    g0_ref[0, 0, :] = jnp.where(keep0, v0, 0.0)
    g1_ref[0, 0, :] = jnp.where(keep1, v1, 0.0)

    g = jax.nn.sigmoid(jax.lax.dot_general(
        x, wg_ref[...], (((1,), (1,)), ((), ())),
        preferred_element_type=jnp.float32))
    h1 = _gelu(jax.lax.dot_general(x, ws1_ref[...],
                                   (((1,), (1,)), ((), ())),
                                   preferred_element_type=jnp.float32))
    sh = jax.lax.dot_general(h1, ws2_ref[...],
                             (((1,), (1,)), ((), ())),
                             preferred_element_type=jnp.float32)
    y_ref[...] = g * sh

    @pl.when(t == NT - 1)
    def _():
        imp_ref[...] = imp_ref[...] * (1.0 / N)
        cnt_ref[...] = cnt_ref[...] * (1.0 / (float(NK) + 1e-12))


def _router_call(x, Wr, Wg, Ws1, Ws2):
    f32 = jnp.float32
    i32 = jnp.int32
    out_shapes = (
        jax.ShapeDtypeStruct((N, D), f32),
        jax.ShapeDtypeStruct((NT, 1, TB), i32),
        jax.ShapeDtypeStruct((NT, 1, TB), i32),
        jax.ShapeDtypeStruct((NT, 1, TB), i32),
        jax.ShapeDtypeStruct((NT, 1, TB), i32),
        jax.ShapeDtypeStruct((NT, 1, TB), f32),
        jax.ShapeDtypeStruct((NT, 1, TB), f32),
        jax.ShapeDtypeStruct((1, E), f32),
        jax.ShapeDtypeStruct((1, E), f32),
    )
    blk3 = pl.BlockSpec((1, 1, TB), lambda t: (t, 0, 0))
    return pl.pallas_call(
        _router_body,
        grid=(NT,),
        in_specs=[
            pl.BlockSpec((TB, D), lambda t: (t, 0)),
            pl.BlockSpec((E, D), lambda t: (0, 0)),
            pl.BlockSpec((1, D), lambda t: (0, 0)),
            pl.BlockSpec((D_FF, D), lambda t: (0, 0)),
            pl.BlockSpec((D, D_FF), lambda t: (0, 0)),
        ],
        out_specs=(
            pl.BlockSpec((TB, D), lambda t: (t, 0)),
            blk3, blk3, blk3, blk3, blk3, blk3,
            pl.BlockSpec((1, E), lambda t: (0, 0)),
            pl.BlockSpec((1, E), lambda t: (0, 0)),
        ),
        out_shape=out_shapes,
        scratch_shapes=[pltpu.VMEM((1, E), f32)],
    )(x, Wr, Wg, Ws1, Ws2)


def _dispatch_body(x_hbm, sd0_hbm, sd1_hbm, disp_hbm,
                   rows_v, s0_v, s1_v, sem0, sem1):
    wid = lax.axis_index("c") * 16 + lax.axis_index("s")
    base = wid * TPW

    def chunk(c, _):
        off = base + c * DCH
        pltpu.sync_copy(sd0_hbm.at[pl.ds(off, DCH)], s0_v)
        pltpu.sync_copy(sd1_hbm.at[pl.ds(off, DCH)], s1_v)
        pltpu.sync_copy(x_hbm.at[pl.ds(off, DCH)], rows_v)
        cp0 = pltpu.async_copy(rows_v, disp_hbm.at[s0_v], sem0)
        cp1 = pltpu.async_copy(rows_v, disp_hbm.at[s1_v], sem1)
        cp0.wait()
        cp1.wait()
        return 0

    lax.fori_loop(0, TPW // DCH, chunk, 0)


def _dispatch_call(x, sd0, sd1):
    mesh = plsc.VectorSubcoreMesh(core_axis_name="c", subcore_axis_name="s")
    kfn = pl.kernel(
        _dispatch_body,
        out_type=jax.ShapeDtypeStruct((DISP_ROWS, D), jnp.float32),
        mesh=mesh,
        scratch_types=[
            pltpu.VMEM((DCH, D), jnp.float32),
            pltpu.VMEM((DCH,), jnp.int32),
            pltpu.VMEM((DCH,), jnp.int32),
            pltpu.SemaphoreType.DMA,
            pltpu.SemaphoreType.DMA,
        ],
    )
    return kfn(x, sd0, sd1)


def _ffn_body(disp_ref, w1_ref, w2_ref, out_ref):
    d = disp_ref[...]
    h = jax.lax.dot_general(d, w1_ref[0], (((1,), (0,)), ((), ())),
                            preferred_element_type=jnp.float32)
    h = _gelu(h)
    out_ref[...] = jax.lax.dot_general(h, w2_ref[0], (((1,), (0,)), ((), ())),
                                       preferred_element_type=jnp.float32)


def _ffn_call(disp, We1, We2):
    return pl.pallas_call(
        _ffn_body,
        grid=(E,),
        in_specs=[
            pl.BlockSpec((CAP, D), lambda e: (e, 0)),
            pl.BlockSpec((1, D, D_EXP), lambda e: (e, 0, 0)),
            pl.BlockSpec((1, D_EXP, D), lambda e: (e, 0, 0)),
        ],
        out_specs=pl.BlockSpec((CAP, D), lambda e: (e, 0)),
        out_shape=jax.ShapeDtypeStruct((E * CAP, D), jnp.float32),
    )(disp, We1, We2)


def _combine_body(eo_hbm, ybase_hbm, sc0_hbm, sc1_hbm, g0_hbm, g1_hbm,
                  y_hbm, r0_v, r1_v, base_v, out_v, s0_v, s1_v,
                  g0_v, g1_v, sem0, sem1, sem2):
    wid = lax.axis_index("c") * 16 + lax.axis_index("s")
    base = wid * TPW

    def chunk(c, _):
        off = base + c * CCH
        pltpu.sync_copy(sc0_hbm.at[pl.ds(off, CCH)], s0_v)
        pltpu.sync_copy(sc1_hbm.at[pl.ds(off, CCH)], s1_v)
        pltpu.sync_copy(g0_hbm.at[pl.ds(off, CCH)], g0_v)
        pltpu.sync_copy(g1_hbm.at[pl.ds(off, CCH)], g1_v)
        cp0 = pltpu.async_copy(eo_hbm.at[s0_v], r0_v, sem0)
        cp1 = pltpu.async_copy(eo_hbm.at[s1_v], r1_v, sem1)
        cp2 = pltpu.async_copy(ybase_hbm.at[pl.ds(off, CCH)], base_v, sem2)
        cp0.wait()
        cp1.wait()
        cp2.wait()

        def token(ti, _):
            idx = jnp.zeros((16,), jnp.int32) + ti
            gv0 = plsc.load_gather(g0_v, [idx])
            gv1 = plsc.load_gather(g1_v, [idx])

            def seg(si, _):
                o = (base_v[ti, pl.ds(si * 16, 16)]
                     + gv0 * r0_v[ti, pl.ds(si * 16, 16)]
                     + gv1 * r1_v[ti, pl.ds(si * 16, 16)])
                out_v[ti, pl.ds(si * 16, 16)] = o
                return 0

            lax.fori_loop(0, D // 16, seg, 0)
            return 0

        lax.fori_loop(0, CCH, token, 0)
        pltpu.sync_copy(out_v, y_hbm.at[pl.ds(off, CCH)])
        return 0

    lax.fori_loop(0, TPW // CCH, chunk, 0)


def _combine_call(eo, y_base, sc0, sc1, g0, g1):
    mesh = plsc.VectorSubcoreMesh(core_axis_name="c", subcore_axis_name="s")
    kfn = pl.kernel(
        _combine_body,
        out_type=jax.ShapeDtypeStruct((N, D), jnp.float32),
        mesh=mesh,
        scratch_types=[
            pltpu.VMEM((CCH, D), jnp.float32),
            pltpu.VMEM((CCH, D), jnp.float32),
            pltpu.VMEM((CCH, D), jnp.float32),
            pltpu.VMEM((CCH, D), jnp.float32),
            pltpu.VMEM((CCH,), jnp.int32),
            pltpu.VMEM((CCH,), jnp.int32),
            pltpu.VMEM((CCH,), jnp.float32),
            pltpu.VMEM((CCH,), jnp.float32),
            pltpu.SemaphoreType.DMA,
            pltpu.SemaphoreType.DMA,
            pltpu.SemaphoreType.DMA,
        ],
    )
    return kfn(eo, y_base, sc0, sc1, g0, g1)


@jax.jit
def kernel(hidden_state, Wr, Wg, We1, We2, Ws1, Ws2):
    x = hidden_state.reshape(N, D)
    (y_base, sd0, sd1, sc0, sc1, g0, g1, imp, cnt) = _router_call(
        x, Wr, Wg, Ws1, Ws2)
    disp = _dispatch_call(x, sd0.reshape(N), sd1.reshape(N))
    eo = _ffn_call(disp, We1, We2)
    y = _combine_call(eo, y_base, sc0.reshape(N), sc1.reshape(N),
                      g0.reshape(N), g1.reshape(N))
    return y.reshape(B, T, D), imp.reshape(E), cnt.reshape(E)


# speedup vs baseline: 2.9981x; 2.9981x over previous
import functools
import math

import jax
import jax.numpy as jnp
from jax import lax
from jax.experimental import pallas as pl
from jax.experimental.pallas import tpu as pltpu
from jax.experimental.pallas import tpu_sc as plsc

B, T, D = 4, 4096, 1024
E, K = 64, 2
D_EXP, D_FF = 256, 256
N = B * T
NK = N * K
CAP = int(math.ceil(N * K / E * 1.25))
TB = 512
NT = N // TB
NWORK = 32
TPW = N // NWORK
DCH = 32
CCH = 16
TRASH = E * CAP
DISP_ROWS = E * CAP + 8


def _gelu(x):
    return 0.5 * x * (1.0 + lax.erf(x * 0.7071067811865476))


def _router_body(x_ref, wr_ref, wg_ref, ws1_ref, ws2_ref,
                 y_ref, sd0_ref, sd1_ref, sc0_ref, sc1_ref,
                 g0_ref, g1_ref, imp_ref, cnt_ref, carry_ref):
    t = pl.program_id(0)

    @pl.when(t == 0)
    def _():
        imp_ref[...] = jnp.zeros_like(imp_ref)
        cnt_ref[...] = jnp.zeros_like(cnt_ref)
        carry_ref[...] = jnp.zeros_like(carry_ref)

    x = x_ref[...]
    logits = jax.lax.dot_general(x, wr_ref[...],
                                 (((1,), (1,)), ((), ())),
                                 preferred_element_type=jnp.float32)
    m = jnp.max(logits, axis=1, keepdims=True)
    ex = jnp.exp(logits - m)
    s = ex / jnp.sum(ex, axis=1, keepdims=True)

    iota_e = lax.broadcasted_iota(jnp.int32, (TB, E), 1)
    v0 = jnp.max(s, axis=1)
    i0 = jnp.min(jnp.where(s == v0[:, None], iota_e, E), axis=1)
    s_m = jnp.where(iota_e == i0[:, None], -1.0, s)
    v1 = jnp.max(s_m, axis=1)
    i1 = jnp.min(jnp.where(s_m == v1[:, None], iota_e, E), axis=1)

    oh0 = (iota_e == i0[:, None]).astype(jnp.float32)
    oh1 = (iota_e == i1[:, None]).astype(jnp.float32)
    ohs = oh0 + oh1

    imp_ref[...] += jnp.sum(s, axis=0)[None, :]
    cnt_ref[...] += jnp.sum(ohs, axis=0)[None, :]

    ri = lax.broadcasted_iota(jnp.int32, (TB, TB), 0)
    ci = lax.broadcasted_iota(jnp.int32, (TB, TB), 1)
    tri = (ri > ci).astype(jnp.float32)
    cum = jax.lax.dot_general(tri, ohs, (((1,), (0,)), ((), ())),
                              preferred_element_type=jnp.float32)
    cum = cum + carry_ref[...]
    p0 = jnp.sum(cum * oh0, axis=1)
    p1 = jnp.sum(cum * oh1, axis=1)
    carry_ref[...] += jnp.sum(ohs, axis=0)[None, :]

    keep0 = p0 < float(CAP)
    keep1 = p1 < float(CAP)
    slot0 = i0 * CAP + p0.astype(jnp.int32)
    slot1 = i1 * CAP + p1.astype(jnp.int32)
    sd0_ref[0, 0, :] = jnp.where(keep0, slot0, TRASH)
    sd1_ref[0, 0, :] = jnp.where(keep1, slot1, TRASH)
    sc0_ref[0, 0, :] = jnp.where(keep0, slot0, i0 * CAP)
    sc1_ref[0, 0, :] = jnp.where(keep1, slot1, i1 * CAP)
    g0_ref[...] = jnp.broadcast_to(jnp.where(keep0, v0, 0.0)[:, None], (TB, 16))
    g1_ref[...] = jnp.broadcast_to(jnp.where(keep1, v1, 0.0)[:, None], (TB, 16))

    g = jax.nn.sigmoid(jax.lax.dot_general(
        x, wg_ref[...], (((1,), (1,)), ((), ())),
        preferred_element_type=jnp.float32))
    h1 = _gelu(jax.lax.dot_general(x, ws1_ref[...],
                                   (((1,), (1,)), ((), ())),
                                   preferred_element_type=jnp.float32))
    sh = jax.lax.dot_general(h1, ws2_ref[...],
                             (((1,), (1,)), ((), ())),
                             preferred_element_type=jnp.float32)
    y_ref[...] = g * sh

    @pl.when(t == NT - 1)
    def _():
        imp_ref[...] = imp_ref[...] * (1.0 / N)
        cnt_ref[...] = cnt_ref[...] * (1.0 / (float(NK) + 1e-12))


def _router_call(x, Wr, Wg, Ws1, Ws2):
    f32 = jnp.float32
    i32 = jnp.int32
    out_shapes = (
        jax.ShapeDtypeStruct((N, D), f32),
        jax.ShapeDtypeStruct((NT, 1, TB), i32),
        jax.ShapeDtypeStruct((NT, 1, TB), i32),
        jax.ShapeDtypeStruct((NT, 1, TB), i32),
        jax.ShapeDtypeStruct((NT, 1, TB), i32),
        jax.ShapeDtypeStruct((N, 16), f32),
        jax.ShapeDtypeStruct((N, 16), f32),
        jax.ShapeDtypeStruct((1, E), f32),
        jax.ShapeDtypeStruct((1, E), f32),
    )
    blk3 = pl.BlockSpec((1, 1, TB), lambda t: (t, 0, 0))
    return pl.pallas_call(
        _router_body,
        grid=(NT,),
        in_specs=[
            pl.BlockSpec((TB, D), lambda t: (t, 0)),
            pl.BlockSpec((E, D), lambda t: (0, 0)),
            pl.BlockSpec((1, D), lambda t: (0, 0)),
            pl.BlockSpec((D_FF, D), lambda t: (0, 0)),
            pl.BlockSpec((D, D_FF), lambda t: (0, 0)),
        ],
        out_specs=(
            pl.BlockSpec((TB, D), lambda t: (t, 0)),
            blk3, blk3, blk3, blk3,
            pl.BlockSpec((TB, 16), lambda t: (t, 0)),
            pl.BlockSpec((TB, 16), lambda t: (t, 0)),
            pl.BlockSpec((1, E), lambda t: (0, 0)),
            pl.BlockSpec((1, E), lambda t: (0, 0)),
        ),
        out_shape=out_shapes,
        scratch_shapes=[pltpu.VMEM((1, E), f32)],
    )(x, Wr, Wg, Ws1, Ws2)


def _dispatch_body(x_hbm, sd0_hbm, sd1_hbm, disp_hbm,
                   rows_v, s0_v, s1_v, sem0, sem1):
    wid = lax.axis_index("c") * 16 + lax.axis_index("s")
    base = wid * TPW

    def chunk(c, _):
        off = base + c * DCH
        pltpu.sync_copy(sd0_hbm.at[pl.ds(off, DCH)], s0_v)
        pltpu.sync_copy(sd1_hbm.at[pl.ds(off, DCH)], s1_v)
        pltpu.sync_copy(x_hbm.at[pl.ds(off, DCH)], rows_v)
        cp0 = pltpu.async_copy(rows_v, disp_hbm.at[s0_v], sem0)
        cp1 = pltpu.async_copy(rows_v, disp_hbm.at[s1_v], sem1)
        cp0.wait()
        cp1.wait()
        return 0

    lax.fori_loop(0, TPW // DCH, chunk, 0)


def _dispatch_call(x, sd0, sd1):
    mesh = plsc.VectorSubcoreMesh(core_axis_name="c", subcore_axis_name="s")
    kfn = pl.kernel(
        _dispatch_body,
        out_type=jax.ShapeDtypeStruct((DISP_ROWS, D), jnp.float32),
        mesh=mesh,
        scratch_types=[
            pltpu.VMEM((DCH, D), jnp.float32),
            pltpu.VMEM((DCH,), jnp.int32),
            pltpu.VMEM((DCH,), jnp.int32),
            pltpu.SemaphoreType.DMA,
            pltpu.SemaphoreType.DMA,
        ],
    )
    return kfn(x, sd0, sd1)


def _ffn_body(disp_ref, w1_ref, w2_ref, out_ref):
    d = disp_ref[...]
    h = jax.lax.dot_general(d, w1_ref[0], (((1,), (0,)), ((), ())),
                            preferred_element_type=jnp.float32)
    h = _gelu(h)
    out_ref[...] = jax.lax.dot_general(h, w2_ref[0], (((1,), (0,)), ((), ())),
                                       preferred_element_type=jnp.float32)


def _ffn_call(disp, We1, We2):
    return pl.pallas_call(
        _ffn_body,
        grid=(E,),
        in_specs=[
            pl.BlockSpec((CAP, D), lambda e: (e, 0)),
            pl.BlockSpec((1, D, D_EXP), lambda e: (e, 0, 0)),
            pl.BlockSpec((1, D_EXP, D), lambda e: (e, 0, 0)),
        ],
        out_specs=pl.BlockSpec((CAP, D), lambda e: (e, 0)),
        out_shape=jax.ShapeDtypeStruct((E * CAP, D), jnp.float32),
    )(disp, We1, We2)


def _combine_body(eo_hbm, ybase_hbm, sc0_hbm, sc1_hbm, g0_hbm, g1_hbm,
                  y_hbm, r0_v, r1_v, base_v, out_v, s0_v, s1_v,
                  g0_v, g1_v, sem0, sem1, sem2):
    wid = lax.axis_index("c") * 16 + lax.axis_index("s")
    base = wid * TPW

    def chunk(c, _):
        off = base + c * CCH
        pltpu.sync_copy(sc0_hbm.at[pl.ds(off, CCH)], s0_v)
        pltpu.sync_copy(sc1_hbm.at[pl.ds(off, CCH)], s1_v)
        pltpu.sync_copy(g0_hbm.at[pl.ds(off, CCH)], g0_v)
        pltpu.sync_copy(g1_hbm.at[pl.ds(off, CCH)], g1_v)
        cp0 = pltpu.async_copy(eo_hbm.at[s0_v], r0_v, sem0)
        cp1 = pltpu.async_copy(eo_hbm.at[s1_v], r1_v, sem1)
        cp2 = pltpu.async_copy(ybase_hbm.at[pl.ds(off, CCH)], base_v, sem2)
        cp0.wait()
        cp1.wait()
        cp2.wait()

        for ti in range(CCH):
            gv0 = g0_v[ti]
            gv1 = g1_v[ti]

            def seg(si, _):
                o = (base_v[ti, pl.ds(si * 16, 16)]
                     + gv0 * r0_v[ti, pl.ds(si * 16, 16)]
                     + gv1 * r1_v[ti, pl.ds(si * 16, 16)])
                out_v[ti, pl.ds(si * 16, 16)] = o
                return 0

            lax.fori_loop(0, D // 16, seg, 0)

        pltpu.sync_copy(out_v, y_hbm.at[pl.ds(off, CCH)])
        return 0

    lax.fori_loop(0, TPW // CCH, chunk, 0)


def _combine_call(eo, y_base, sc0, sc1, g0, g1):
    mesh = plsc.VectorSubcoreMesh(core_axis_name="c", subcore_axis_name="s")
    kfn = pl.kernel(
        _combine_body,
        out_type=jax.ShapeDtypeStruct((N, D), jnp.float32),
        mesh=mesh,
        scratch_types=[
            pltpu.VMEM((CCH, D), jnp.float32),
            pltpu.VMEM((CCH, D), jnp.float32),
            pltpu.VMEM((CCH, D), jnp.float32),
            pltpu.VMEM((CCH, D), jnp.float32),
            pltpu.VMEM((CCH,), jnp.int32),
            pltpu.VMEM((CCH,), jnp.int32),
            pltpu.VMEM((CCH, 16), jnp.float32),
            pltpu.VMEM((CCH, 16), jnp.float32),
            pltpu.SemaphoreType.DMA,
            pltpu.SemaphoreType.DMA,
            pltpu.SemaphoreType.DMA,
        ],
    )
    return kfn(eo, y_base, sc0, sc1, g0, g1)


@jax.jit
def kernel(hidden_state, Wr, Wg, We1, We2, Ws1, Ws2):
    x = hidden_state.reshape(N, D)
    (y_base, sd0, sd1, sc0, sc1, g0, g1, imp, cnt) = _router_call(
        x, Wr, Wg, Ws1, Ws2)
    disp = _dispatch_call(x, sd0.reshape(N), sd1.reshape(N))
    eo = _ffn_call(disp, We1, We2)
    y = _combine_call(eo, y_base, sc0.reshape(N), sc1.reshape(N), g0, g1)
    return y.reshape(B, T, D), imp.reshape(E), cnt.reshape(E)


# speedup vs baseline: 3.5499x; 1.1841x over previous
import functools
import math

import jax
import jax.numpy as jnp
from jax import lax
from jax.experimental import pallas as pl
from jax.experimental.pallas import tpu as pltpu
from jax.experimental.pallas import tpu_sc as plsc

B, T, D = 4, 4096, 1024
E, K = 64, 2
D_EXP, D_FF = 256, 256
N = B * T
NK = N * K
CAP = int(math.ceil(N * K / E * 1.25))
TB = 512
NT = N // TB
NWORK = 32
TPW = N // NWORK
DCH = 32
CCH = 16
TRASH = E * CAP
DISP_ROWS = E * CAP + 8


def _gelu(x):
    return 0.5 * x * (1.0 + lax.erf(x * 0.7071067811865476))


def _router_body(x_ref, wr_ref, wg_ref, ws1_ref, ws2_ref,
                 y_ref, sd0_ref, sd1_ref, sc0_ref, sc1_ref,
                 g0_ref, g1_ref, imp_ref, cnt_ref, carry_ref):
    t = pl.program_id(0)

    @pl.when(t == 0)
    def _():
        imp_ref[...] = jnp.zeros_like(imp_ref)
        cnt_ref[...] = jnp.zeros_like(cnt_ref)
        carry_ref[...] = jnp.zeros_like(carry_ref)

    x = x_ref[...]
    logits = jax.lax.dot_general(x, wr_ref[...],
                                 (((1,), (1,)), ((), ())),
                                 preferred_element_type=jnp.float32)
    m = jnp.max(logits, axis=1, keepdims=True)
    ex = jnp.exp(logits - m)
    s = ex / jnp.sum(ex, axis=1, keepdims=True)

    iota_e = lax.broadcasted_iota(jnp.int32, (TB, E), 1)
    v0 = jnp.max(s, axis=1)
    i0 = jnp.min(jnp.where(s == v0[:, None], iota_e, E), axis=1)
    s_m = jnp.where(iota_e == i0[:, None], -1.0, s)
    v1 = jnp.max(s_m, axis=1)
    i1 = jnp.min(jnp.where(s_m == v1[:, None], iota_e, E), axis=1)

    oh0 = (iota_e == i0[:, None]).astype(jnp.float32)
    oh1 = (iota_e == i1[:, None]).astype(jnp.float32)
    ohs = oh0 + oh1

    imp_ref[...] += jnp.sum(s, axis=0)[None, :]
    cnt_ref[...] += jnp.sum(ohs, axis=0)[None, :]

    ri = lax.broadcasted_iota(jnp.int32, (TB, TB), 0)
    ci = lax.broadcasted_iota(jnp.int32, (TB, TB), 1)
    tri = (ri > ci).astype(jnp.float32)
    cum = jax.lax.dot_general(tri, ohs, (((1,), (0,)), ((), ())),
                              preferred_element_type=jnp.float32)
    cum = cum + carry_ref[...]
    p0 = jnp.sum(cum * oh0, axis=1)
    p1 = jnp.sum(cum * oh1, axis=1)
    carry_ref[...] += jnp.sum(ohs, axis=0)[None, :]

    keep0 = p0 < float(CAP)
    keep1 = p1 < float(CAP)
    slot0 = i0 * CAP + p0.astype(jnp.int32)
    slot1 = i1 * CAP + p1.astype(jnp.int32)
    sd0_ref[0, 0, :] = jnp.where(keep0, slot0, TRASH)
    sd1_ref[0, 0, :] = jnp.where(keep1, slot1, TRASH)
    sc0_ref[0, 0, :] = jnp.where(keep0, slot0, i0 * CAP)
    sc1_ref[0, 0, :] = jnp.where(keep1, slot1, i1 * CAP)
    g0_ref[...] = jnp.broadcast_to(jnp.where(keep0, v0, 0.0)[:, None], (TB, 16))
    g1_ref[...] = jnp.broadcast_to(jnp.where(keep1, v1, 0.0)[:, None], (TB, 16))

    g = jax.nn.sigmoid(jax.lax.dot_general(
        x, wg_ref[...], (((1,), (1,)), ((), ())),
        preferred_element_type=jnp.float32))
    h1 = _gelu(jax.lax.dot_general(x, ws1_ref[...],
                                   (((1,), (1,)), ((), ())),
                                   preferred_element_type=jnp.float32))
    sh = jax.lax.dot_general(h1, ws2_ref[...],
                             (((1,), (1,)), ((), ())),
                             preferred_element_type=jnp.float32)
    y_ref[...] = g * sh

    @pl.when(t == NT - 1)
    def _():
        imp_ref[...] = imp_ref[...] * (1.0 / N)
        cnt_ref[...] = cnt_ref[...] * (1.0 / (float(NK) + 1e-12))


def _router_call(x, Wr, Wg, Ws1, Ws2):
    f32 = jnp.float32
    i32 = jnp.int32
    out_shapes = (
        jax.ShapeDtypeStruct((N, D), f32),
        jax.ShapeDtypeStruct((NT, 1, TB), i32),
        jax.ShapeDtypeStruct((NT, 1, TB), i32),
        jax.ShapeDtypeStruct((NT, 1, TB), i32),
        jax.ShapeDtypeStruct((NT, 1, TB), i32),
        jax.ShapeDtypeStruct((N, 16), f32),
        jax.ShapeDtypeStruct((N, 16), f32),
        jax.ShapeDtypeStruct((1, E), f32),
        jax.ShapeDtypeStruct((1, E), f32),
    )
    blk3 = pl.BlockSpec((1, 1, TB), lambda t: (t, 0, 0))
    return pl.pallas_call(
        _router_body,
        grid=(NT,),
        in_specs=[
            pl.BlockSpec((TB, D), lambda t: (t, 0)),
            pl.BlockSpec((E, D), lambda t: (0, 0)),
            pl.BlockSpec((1, D), lambda t: (0, 0)),
            pl.BlockSpec((D_FF, D), lambda t: (0, 0)),
            pl.BlockSpec((D, D_FF), lambda t: (0, 0)),
        ],
        out_specs=(
            pl.BlockSpec((TB, D), lambda t: (t, 0)),
            blk3, blk3, blk3, blk3,
            pl.BlockSpec((TB, 16), lambda t: (t, 0)),
            pl.BlockSpec((TB, 16), lambda t: (t, 0)),
            pl.BlockSpec((1, E), lambda t: (0, 0)),
            pl.BlockSpec((1, E), lambda t: (0, 0)),
        ),
        out_shape=out_shapes,
        scratch_shapes=[pltpu.VMEM((1, E), f32)],
    )(x, Wr, Wg, Ws1, Ws2)


def _dispatch_body(x_hbm, sd0_hbm, sd1_hbm, disp_hbm,
                   rows_v, s0_v, s1_v, sem0, sem1):
    wid = lax.axis_index("c") * 16 + lax.axis_index("s")
    base = wid * TPW

    def chunk(c, _):
        off = base + c * DCH
        pltpu.sync_copy(sd0_hbm.at[pl.ds(off, DCH)], s0_v)
        pltpu.sync_copy(sd1_hbm.at[pl.ds(off, DCH)], s1_v)
        pltpu.sync_copy(x_hbm.at[pl.ds(off, DCH)], rows_v)
        cp0 = pltpu.async_copy(rows_v, disp_hbm.at[s0_v], sem0)
        cp1 = pltpu.async_copy(rows_v, disp_hbm.at[s1_v], sem1)
        cp0.wait()
        cp1.wait()
        return 0

    lax.fori_loop(0, TPW // DCH, chunk, 0)


def _dispatch_call(x, sd0, sd1):
    mesh = plsc.VectorSubcoreMesh(core_axis_name="c", subcore_axis_name="s")
    kfn = pl.kernel(
        _dispatch_body,
        out_type=jax.ShapeDtypeStruct((DISP_ROWS, D), jnp.float32),
        mesh=mesh,
        scratch_types=[
            pltpu.VMEM((DCH, D), jnp.float32),
            pltpu.VMEM((DCH,), jnp.int32),
            pltpu.VMEM((DCH,), jnp.int32),
            pltpu.SemaphoreType.DMA,
            pltpu.SemaphoreType.DMA,
        ],
    )
    return kfn(x, sd0, sd1)


def _ffn_body(disp_ref, w1_ref, w2_ref, out_ref):
    d = disp_ref[...]
    h = jax.lax.dot_general(d, w1_ref[0], (((1,), (0,)), ((), ())),
                            preferred_element_type=jnp.float32)
    h = _gelu(h)
    out_ref[...] = jax.lax.dot_general(h, w2_ref[0], (((1,), (0,)), ((), ())),
                                       preferred_element_type=jnp.float32)


def _ffn_call(disp, We1, We2):
    return pl.pallas_call(
        _ffn_body,
        grid=(E,),
        in_specs=[
            pl.BlockSpec((CAP, D), lambda e: (e, 0)),
            pl.BlockSpec((1, D, D_EXP), lambda e: (e, 0, 0)),
            pl.BlockSpec((1, D_EXP, D), lambda e: (e, 0, 0)),
        ],
        out_specs=pl.BlockSpec((CAP, D), lambda e: (e, 0)),
        out_shape=jax.ShapeDtypeStruct((E * CAP, D), jnp.float32),
    )(disp, We1, We2)


def _combine_body(eo_hbm, ybase_hbm, sc0_hbm, sc1_hbm, g0_hbm, g1_hbm,
                  y_hbm,
                  r0a_v, r1a_v, acca_v, s0a_v, s1a_v, g0a_v, g1a_v,
                  r0b_v, r1b_v, accb_v, s0b_v, s1b_v, g0b_v, g1b_v,
                  sem0a, sem1a, sem2a, sem0b, sem1b, sem2b):
    wid = lax.axis_index("c") * 16 + lax.axis_index("s")
    base = wid * TPW
    bufs = ((r0a_v, r1a_v, acca_v, s0a_v, s1a_v, g0a_v, g1a_v,
             sem0a, sem1a, sem2a),
            (r0b_v, r1b_v, accb_v, s0b_v, s1b_v, g0b_v, g1b_v,
             sem0b, sem1b, sem2b))

    def start(off, buf):
        r0_v, r1_v, acc_v, s0_v, s1_v, g0_v, g1_v, sm0, sm1, sm2 = buf
        pltpu.sync_copy(sc0_hbm.at[pl.ds(off, CCH)], s0_v)
        pltpu.sync_copy(sc1_hbm.at[pl.ds(off, CCH)], s1_v)
        pltpu.sync_copy(g0_hbm.at[pl.ds(off, CCH)], g0_v)
        pltpu.sync_copy(g1_hbm.at[pl.ds(off, CCH)], g1_v)
        cp0 = pltpu.async_copy(eo_hbm.at[s0_v], r0_v, sm0)
        cp1 = pltpu.async_copy(eo_hbm.at[s1_v], r1_v, sm1)
        cp2 = pltpu.async_copy(ybase_hbm.at[pl.ds(off, CCH)], acc_v, sm2)
        return cp0, cp1, cp2

    def finish(off, buf, cps):
        r0_v, r1_v, acc_v = buf[0], buf[1], buf[2]
        g0_v, g1_v = buf[5], buf[6]
        for cp in cps:
            cp.wait()
        for ti in range(CCH):
            gv0 = g0_v[ti]
            gv1 = g1_v[ti]

            def seg(si, _):
                for u in range(8):
                    d = pl.ds(si * 128 + u * 16, 16)
                    acc_v[ti, d] += gv0 * r0_v[ti, d] + gv1 * r1_v[ti, d]
                return 0

            lax.fori_loop(0, D // 128, seg, 0)
        pltpu.sync_copy(acc_v, y_hbm.at[pl.ds(off, CCH)])

    def pair(i, _):
        offa = base + i * (2 * CCH)
        offb = offa + CCH
        cpa = start(offa, bufs[0])
        cpb = start(offb, bufs[1])
        finish(offa, bufs[0], cpa)
        finish(offb, bufs[1], cpb)
        return 0

    lax.fori_loop(0, TPW // (2 * CCH), pair, 0)


def _combine_call(eo, y_base, sc0, sc1, g0, g1):
    mesh = plsc.VectorSubcoreMesh(core_axis_name="c", subcore_axis_name="s")
    bufset = [
        pltpu.VMEM((CCH, D), jnp.float32),
        pltpu.VMEM((CCH, D), jnp.float32),
        pltpu.VMEM((CCH, D), jnp.float32),
        pltpu.VMEM((CCH,), jnp.int32),
        pltpu.VMEM((CCH,), jnp.int32),
        pltpu.VMEM((CCH, 16), jnp.float32),
        pltpu.VMEM((CCH, 16), jnp.float32),
    ]
    kfn = pl.kernel(
        _combine_body,
        out_type=jax.ShapeDtypeStruct((N, D), jnp.float32),
        mesh=mesh,
        scratch_types=bufset + bufset + [pltpu.SemaphoreType.DMA] * 6,
    )
    return kfn(eo, y_base, sc0, sc1, g0, g1)


@jax.jit
def kernel(hidden_state, Wr, Wg, We1, We2, Ws1, Ws2):
    x = hidden_state.reshape(N, D)
    (y_base, sd0, sd1, sc0, sc1, g0, g1, imp, cnt) = _router_call(
        x, Wr, Wg, Ws1, Ws2)
    disp = _dispatch_call(x, sd0.reshape(N), sd1.reshape(N))
    eo = _ffn_call(disp, We1, We2)
    y = _combine_call(eo, y_base, sc0.reshape(N), sc1.reshape(N), g0, g1)
    return y.reshape(B, T, D), imp.reshape(E), cnt.reshape(E)


# speedup vs baseline: 3.6723x; 1.0345x over previous
import functools
import math

import jax
import jax.numpy as jnp
from jax import lax
from jax.experimental import pallas as pl
from jax.experimental.pallas import tpu as pltpu
from jax.experimental.pallas import tpu_sc as plsc

B, T, D = 4, 4096, 1024
E, K = 64, 2
D_EXP, D_FF = 256, 256
N = B * T
NK = N * K
CAP = int(math.ceil(N * K / E * 1.25))
TB = 512
NT = N // TB
NWORK = 32
TPW = N // NWORK
DCH = 32
CCH = 16
TRASH = E * CAP
DISP_ROWS = E * CAP + 8


def _gelu(x):
    return 0.5 * x * (1.0 + lax.erf(x * 0.7071067811865476))


def _router_body(x_ref, wr_ref, wg_ref, ws1_ref, ws2_ref,
                 y_ref, sd0_ref, sd1_ref, sc0_ref, sc1_ref,
                 g0_ref, g1_ref, imp_ref, cnt_ref, carry_ref):
    t = pl.program_id(0)

    @pl.when(t == 0)
    def _():
        imp_ref[...] = jnp.zeros_like(imp_ref)
        cnt_ref[...] = jnp.zeros_like(cnt_ref)
        carry_ref[...] = jnp.zeros_like(carry_ref)

    x = x_ref[...]
    logits = jax.lax.dot_general(x, wr_ref[...],
                                 (((1,), (1,)), ((), ())),
                                 preferred_element_type=jnp.float32)
    m = jnp.max(logits, axis=1, keepdims=True)
    ex = jnp.exp(logits - m)
    s = ex / jnp.sum(ex, axis=1, keepdims=True)

    iota_e = lax.broadcasted_iota(jnp.int32, (TB, E), 1)
    v0 = jnp.max(s, axis=1)
    i0 = jnp.min(jnp.where(s == v0[:, None], iota_e, E), axis=1)
    s_m = jnp.where(iota_e == i0[:, None], -1.0, s)
    v1 = jnp.max(s_m, axis=1)
    i1 = jnp.min(jnp.where(s_m == v1[:, None], iota_e, E), axis=1)

    oh0 = (iota_e == i0[:, None]).astype(jnp.float32)
    oh1 = (iota_e == i1[:, None]).astype(jnp.float32)
    ohs = oh0 + oh1

    imp_ref[...] += jnp.sum(s, axis=0)[None, :]
    cnt_ref[...] += jnp.sum(ohs, axis=0)[None, :]

    ri = lax.broadcasted_iota(jnp.int32, (TB, TB), 0)
    ci = lax.broadcasted_iota(jnp.int32, (TB, TB), 1)
    tri = (ri > ci).astype(jnp.float32)
    cum = jax.lax.dot_general(tri, ohs, (((1,), (0,)), ((), ())),
                              preferred_element_type=jnp.float32)
    cum = cum + carry_ref[...]
    p0 = jnp.sum(cum * oh0, axis=1)
    p1 = jnp.sum(cum * oh1, axis=1)
    carry_ref[...] += jnp.sum(ohs, axis=0)[None, :]

    keep0 = p0 < float(CAP)
    keep1 = p1 < float(CAP)
    slot0 = i0 * CAP + p0.astype(jnp.int32)
    slot1 = i1 * CAP + p1.astype(jnp.int32)
    sd0_ref[0, 0, :] = jnp.where(keep0, slot0, TRASH)
    sd1_ref[0, 0, :] = jnp.where(keep1, slot1, TRASH)
    sc0_ref[0, 0, :] = jnp.where(keep0, slot0, i0 * CAP)
    sc1_ref[0, 0, :] = jnp.where(keep1, slot1, i1 * CAP)
    g0_ref[...] = jnp.broadcast_to(jnp.where(keep0, v0, 0.0)[:, None], (TB, 16))
    g1_ref[...] = jnp.broadcast_to(jnp.where(keep1, v1, 0.0)[:, None], (TB, 16))

    g = jax.nn.sigmoid(jax.lax.dot_general(
        x, wg_ref[...], (((1,), (1,)), ((), ())),
        preferred_element_type=jnp.float32))
    h1 = _gelu(jax.lax.dot_general(x, ws1_ref[...],
                                   (((1,), (1,)), ((), ())),
                                   preferred_element_type=jnp.float32))
    sh = jax.lax.dot_general(h1, ws2_ref[...],
                             (((1,), (1,)), ((), ())),
                             preferred_element_type=jnp.float32)
    y_ref[...] = g * sh

    @pl.when(t == NT - 1)
    def _():
        imp_ref[...] = imp_ref[...] * (1.0 / N)
        cnt_ref[...] = cnt_ref[...] * (1.0 / (float(NK) + 1e-12))


def _router_call(x, Wr, Wg, Ws1, Ws2):
    f32 = jnp.float32
    i32 = jnp.int32
    out_shapes = (
        jax.ShapeDtypeStruct((N, D), f32),
        jax.ShapeDtypeStruct((NT, 1, TB), i32),
        jax.ShapeDtypeStruct((NT, 1, TB), i32),
        jax.ShapeDtypeStruct((NT, 1, TB), i32),
        jax.ShapeDtypeStruct((NT, 1, TB), i32),
        jax.ShapeDtypeStruct((N, 16), f32),
        jax.ShapeDtypeStruct((N, 16), f32),
        jax.ShapeDtypeStruct((1, E), f32),
        jax.ShapeDtypeStruct((1, E), f32),
    )
    blk3 = pl.BlockSpec((1, 1, TB), lambda t: (t, 0, 0))
    return pl.pallas_call(
        _router_body,
        grid=(NT,),
        in_specs=[
            pl.BlockSpec((TB, D), lambda t: (t, 0)),
            pl.BlockSpec((E, D), lambda t: (0, 0)),
            pl.BlockSpec((1, D), lambda t: (0, 0)),
            pl.BlockSpec((D_FF, D), lambda t: (0, 0)),
            pl.BlockSpec((D, D_FF), lambda t: (0, 0)),
        ],
        out_specs=(
            pl.BlockSpec((TB, D), lambda t: (t, 0)),
            blk3, blk3, blk3, blk3,
            pl.BlockSpec((TB, 16), lambda t: (t, 0)),
            pl.BlockSpec((TB, 16), lambda t: (t, 0)),
            pl.BlockSpec((1, E), lambda t: (0, 0)),
            pl.BlockSpec((1, E), lambda t: (0, 0)),
        ),
        out_shape=out_shapes,
        scratch_shapes=[pltpu.VMEM((1, E), f32)],
    )(x, Wr, Wg, Ws1, Ws2)


def _dispatch_body(x_hbm, sd0_hbm, sd1_hbm, disp_hbm,
                   rowsa_v, s0a_v, s1a_v, rowsb_v, s0b_v, s1b_v,
                   semra, sem0a, sem1a, semrb, sem0b, sem1b):
    wid = lax.axis_index("c") * 16 + lax.axis_index("s")
    base = wid * TPW
    bufs = ((rowsa_v, s0a_v, s1a_v, semra, sem0a, sem1a),
            (rowsb_v, s0b_v, s1b_v, semrb, sem0b, sem1b))

    def load(off, buf):
        rows_v, s0_v, s1_v, semr = buf[0], buf[1], buf[2], buf[3]
        pltpu.sync_copy(sd0_hbm.at[pl.ds(off, DCH)], s0_v)
        pltpu.sync_copy(sd1_hbm.at[pl.ds(off, DCH)], s1_v)
        return pltpu.async_copy(x_hbm.at[pl.ds(off, DCH)], rows_v, semr)

    def scatter(buf, cpr):
        rows_v, s0_v, s1_v = buf[0], buf[1], buf[2]
        cpr.wait()
        cp0 = pltpu.async_copy(rows_v, disp_hbm.at[s0_v], buf[4])
        cp1 = pltpu.async_copy(rows_v, disp_hbm.at[s1_v], buf[5])
        return cp0, cp1

    def pair(i, _):
        offa = base + i * (2 * DCH)
        cra = load(offa, bufs[0])
        crb = load(offa + DCH, bufs[1])
        cpa = scatter(bufs[0], cra)
        cpb = scatter(bufs[1], crb)
        for cp in cpa + cpb:
            cp.wait()
        return 0

    lax.fori_loop(0, TPW // (2 * DCH), pair, 0)


def _dispatch_call(x, sd0, sd1):
    mesh = plsc.VectorSubcoreMesh(core_axis_name="c", subcore_axis_name="s")
    bufset = [
        pltpu.VMEM((DCH, D), jnp.float32),
        pltpu.VMEM((DCH,), jnp.int32),
        pltpu.VMEM((DCH,), jnp.int32),
    ]
    kfn = pl.kernel(
        _dispatch_body,
        out_type=jax.ShapeDtypeStruct((DISP_ROWS, D), jnp.float32),
        mesh=mesh,
        scratch_types=bufset + bufset + [pltpu.SemaphoreType.DMA] * 6,
    )
    return kfn(x, sd0, sd1)


def _ffn_body(disp_ref, w1_ref, w2_ref, out_ref):
    d = disp_ref[...]
    h = jax.lax.dot_general(d, w1_ref[0], (((1,), (0,)), ((), ())),
                            preferred_element_type=jnp.float32)
    h = _gelu(h)
    out_ref[...] = jax.lax.dot_general(h, w2_ref[0], (((1,), (0,)), ((), ())),
                                       preferred_element_type=jnp.float32)


def _ffn_call(disp, We1, We2):
    return pl.pallas_call(
        _ffn_body,
        grid=(E,),
        in_specs=[
            pl.BlockSpec((CAP, D), lambda e: (e, 0)),
            pl.BlockSpec((1, D, D_EXP), lambda e: (e, 0, 0)),
            pl.BlockSpec((1, D_EXP, D), lambda e: (e, 0, 0)),
        ],
        out_specs=pl.BlockSpec((CAP, D), lambda e: (e, 0)),
        out_shape=jax.ShapeDtypeStruct((E * CAP, D), jnp.float32),
    )(disp, We1, We2)


def _combine_body(eo_hbm, ybase_hbm, sc0_hbm, sc1_hbm, g0_hbm, g1_hbm,
                  y_hbm,
                  r0a_v, r1a_v, acca_v, s0a_v, s1a_v, g0a_v, g1a_v,
                  r0b_v, r1b_v, accb_v, s0b_v, s1b_v, g0b_v, g1b_v,
                  sem0a, sem1a, sem2a, sem0b, sem1b, sem2b):
    wid = lax.axis_index("c") * 16 + lax.axis_index("s")
    base = wid * TPW
    bufs = ((r0a_v, r1a_v, acca_v, s0a_v, s1a_v, g0a_v, g1a_v,
             sem0a, sem1a, sem2a),
            (r0b_v, r1b_v, accb_v, s0b_v, s1b_v, g0b_v, g1b_v,
             sem0b, sem1b, sem2b))

    def start(off, buf):
        r0_v, r1_v, acc_v, s0_v, s1_v, g0_v, g1_v, sm0, sm1, sm2 = buf
        pltpu.sync_copy(sc0_hbm.at[pl.ds(off, CCH)], s0_v)
        pltpu.sync_copy(sc1_hbm.at[pl.ds(off, CCH)], s1_v)
        pltpu.sync_copy(g0_hbm.at[pl.ds(off, CCH)], g0_v)
        pltpu.sync_copy(g1_hbm.at[pl.ds(off, CCH)], g1_v)
        cp0 = pltpu.async_copy(eo_hbm.at[s0_v], r0_v, sm0)
        cp1 = pltpu.async_copy(eo_hbm.at[s1_v], r1_v, sm1)
        cp2 = pltpu.async_copy(ybase_hbm.at[pl.ds(off, CCH)], acc_v, sm2)
        return cp0, cp1, cp2

    def finish(off, buf, cps):
        r0_v, r1_v, acc_v = buf[0], buf[1], buf[2]
        g0_v, g1_v = buf[5], buf[6]
        for cp in cps:
            cp.wait()
        for ti in range(CCH):
            gv0 = g0_v[ti]
            gv1 = g1_v[ti]

            def seg(si, _):
                for u in range(8):
                    d = pl.ds(si * 128 + u * 16, 16)
                    acc_v[ti, d] += gv0 * r0_v[ti, d] + gv1 * r1_v[ti, d]
                return 0

            lax.fori_loop(0, D // 128, seg, 0)
        pltpu.sync_copy(acc_v, y_hbm.at[pl.ds(off, CCH)])

    def pair(i, _):
        offa = base + i * (2 * CCH)
        offb = offa + CCH
        cpa = start(offa, bufs[0])
        cpb = start(offb, bufs[1])
        finish(offa, bufs[0], cpa)
        finish(offb, bufs[1], cpb)
        return 0

    lax.fori_loop(0, TPW // (2 * CCH), pair, 0)


def _combine_call(eo, y_base, sc0, sc1, g0, g1):
    mesh = plsc.VectorSubcoreMesh(core_axis_name="c", subcore_axis_name="s")
    bufset = [
        pltpu.VMEM((CCH, D), jnp.float32),
        pltpu.VMEM((CCH, D), jnp.float32),
        pltpu.VMEM((CCH, D), jnp.float32),
        pltpu.VMEM((CCH,), jnp.int32),
        pltpu.VMEM((CCH,), jnp.int32),
        pltpu.VMEM((CCH, 16), jnp.float32),
        pltpu.VMEM((CCH, 16), jnp.float32),
    ]
    kfn = pl.kernel(
        _combine_body,
        out_type=jax.ShapeDtypeStruct((N, D), jnp.float32),
        mesh=mesh,
        scratch_types=bufset + bufset + [pltpu.SemaphoreType.DMA] * 6,
    )
    return kfn(eo, y_base, sc0, sc1, g0, g1)


@jax.jit
def kernel(hidden_state, Wr, Wg, We1, We2, Ws1, Ws2):
    x = hidden_state.reshape(N, D)
    (y_base, sd0, sd1, sc0, sc1, g0, g1, imp, cnt) = _router_call(
        x, Wr, Wg, Ws1, Ws2)
    disp = _dispatch_call(x, sd0.reshape(N), sd1.reshape(N))
    eo = _ffn_call(disp, We1, We2)
    y = _combine_call(eo, y_base, sc0.reshape(N), sc1.reshape(N), g0, g1)
    return y.reshape(B, T, D), imp.reshape(E), cnt.reshape(E)
